# Initial kernel scaffold; baseline (speedup 1.0000x reference)
import jax
import jax.numpy as jnp
from jax.experimental import pallas as pl


def kernel(TE, day_table, week_table):
    raise NotImplementedError("write your pallas kernel here")


# speedup vs baseline: 1.1724x; 1.1724x over previous
import functools

import jax
import jax.numpy as jnp
from jax import lax
from jax.experimental import pallas as pl
from jax.experimental.pallas import tpu as pltpu
from jax.experimental.pallas import tpu_sc as plsc

B, T, TDIM = 4096, 200, 64
N = B * T
NC, NS, L = 2, 16, 16
NW = NC * NS
NPW = N // NW
CHUNK = 512
NCHUNK = NPW // CHUNK
SUB = CHUNK // 128
GROUPS = CHUNK // L


def _body(te_hbm, day_hbm, week_hbm, out_hbm,
          te_v, widx_v, didx_v, wrows_v, drows_v, gsem, osem):
    wid = lax.axis_index("s") * NC + lax.axis_index("c")
    lane5 = jax.lax.iota(jnp.int32, L) * 5

    def chunk_body(k, _):
        base = wid * NPW + k * CHUNK
        pltpu.sync_copy(
            te_hbm.at[pl.ds(base * 5 // 128, CHUNK * 5 // 128)], te_v)

        for g in range(GROUPS):
            i5 = lane5 + (g * L * 5)

            def col(off):
                idx = i5 + off
                return plsc.load_gather(
                    te_v, [idx >> 7, idx & 127])

            w = col(2)
            h = col(3)
            m = col(4)
            week = w % 7
            day = ((h % 24) * 60 + (m % 60)) // 5
            widx_v[g // 8, pl.ds((g % 8) * L, L)] = week
            didx_v[g // 8, pl.ds((g % 8) * L, L)] = day

        gathers = []
        for j in range(SUB):
            gathers.append(pltpu.async_copy(
                week_hbm.at[widx_v.at[j]], wrows_v.at[j], gsem))
            gathers.append(pltpu.async_copy(
                day_hbm.at[didx_v.at[j]], drows_v.at[j], gsem))
        for cp in gathers:
            cp.wait()

        writes = []
        for j in range(SUB):
            row = base + j * 128
            writes.append(pltpu.async_copy(
                wrows_v.at[j], out_hbm.at[pl.ds(row, 128), 0], osem))
            writes.append(pltpu.async_copy(
                drows_v.at[j], out_hbm.at[pl.ds(row, 128), 1], osem))
        for cp in writes:
            cp.wait()
        return ()

    lax.fori_loop(0, NCHUNK, chunk_body, (), unroll=False)


@functools.partial(jax.jit, static_argnames=())
def kernel(TE, day_table, week_table):
    te_flat = TE.reshape(N * 5 // 128, 128).astype(jnp.int32)
    run = pl.kernel(
        _body,
        out_type=jax.ShapeDtypeStruct((N, 2, TDIM), jnp.float32),
        mesh=plsc.VectorSubcoreMesh(core_axis_name="c", subcore_axis_name="s"),
        scratch_types=[
            pltpu.VMEM((CHUNK * 5 // 128, 128), jnp.int32),
            pltpu.VMEM((SUB, 128), jnp.int32),
            pltpu.VMEM((SUB, 128), jnp.int32),
            pltpu.VMEM((SUB, 128, TDIM), jnp.float32),
            pltpu.VMEM((SUB, 128, TDIM), jnp.float32),
            pltpu.SemaphoreType.DMA,
            pltpu.SemaphoreType.DMA,
        ],
        compiler_params=pltpu.CompilerParams(
            use_tc_tiling_on_sc=False, needs_layout_passes=False),
    )
    out = run(te_flat, day_table, week_table)
    return out.reshape(B, T, 2 * TDIM)


# speedup vs baseline: 1.1781x; 1.0048x over previous
import functools

import jax
import jax.numpy as jnp
from jax import lax
from jax.experimental import pallas as pl
from jax.experimental.pallas import tpu as pltpu
from jax.experimental.pallas import tpu_sc as plsc

B, T, TDIM = 4096, 200, 64
N = B * T
NC, NS, L = 2, 16, 16
NW = NC * NS
NPW = N // NW
CHUNK = 256
NCHUNK = NPW // CHUNK
SUB = CHUNK // 128
GROUPS = CHUNK // L
TEROWS = CHUNK * 5 // 128


def _body(te_hbm, day_hbm, week_hbm, out_hbm,
          te_v, widx_v, didx_v, wrows_v, drows_v,
          tsem, gsem, osem):
    wid = lax.axis_index("s") * NC + lax.axis_index("c")
    lane5 = jax.lax.iota(jnp.int32, L) * 5
    chunk0 = wid * NCHUNK

    def te_copy(k, slot):
        return pltpu.make_async_copy(
            te_hbm.at[pl.ds((chunk0 + k) * TEROWS, TEROWS)],
            te_v.at[slot], tsem.at[slot])

    def write_copies(k, slot):
        base = (chunk0 + k) * CHUNK
        cps = []
        for j in range(SUB):
            row = base + j * 128
            cps.append(pltpu.make_async_copy(
                wrows_v.at[slot, j], out_hbm.at[pl.ds(row, 128), 0],
                osem.at[slot]))
            cps.append(pltpu.make_async_copy(
                drows_v.at[slot, j], out_hbm.at[pl.ds(row, 128), 1],
                osem.at[slot]))
        return cps

    te_copy(0, 0).start()

    def chunk_step(k, slot):
        te_copy(k, slot).wait()

        @pl.when(k + 1 < NCHUNK)
        def _prefetch():
            te_copy(k + 1, slot ^ 1).start()

        for g in range(GROUPS):
            i5 = lane5 + (g * L * 5)

            def col(off):
                idx = i5 + off
                return plsc.load_gather(te_v.at[slot], [idx >> 7, idx & 127])

            w = col(2)
            h = col(3)
            m = col(4)
            week = w % 7
            day = ((h % 24) * 60 + (m % 60)) // 5
            widx_v[slot, g // 8, pl.ds((g % 8) * L, L)] = week
            didx_v[slot, g // 8, pl.ds((g % 8) * L, L)] = day

        @pl.when(k >= 2)
        def _drain_writes():
            for cp in write_copies(k - 2, slot):
                cp.wait()

        gathers = []
        for j in range(SUB):
            gathers.append(pltpu.make_async_copy(
                week_hbm.at[widx_v.at[slot, j]], wrows_v.at[slot, j],
                gsem.at[slot]))
            gathers.append(pltpu.make_async_copy(
                day_hbm.at[didx_v.at[slot, j]], drows_v.at[slot, j],
                gsem.at[slot]))
        for cp in gathers:
            cp.start()
        for cp in gathers:
            cp.wait()

        for cp in write_copies(k, slot):
            cp.start()

    def pair_step(k2, _):
        chunk_step(2 * k2, 0)
        chunk_step(2 * k2 + 1, 1)
        return ()

    lax.fori_loop(0, NCHUNK // 2, pair_step, (), unroll=False)

    for cp in write_copies(NCHUNK - 2, 0):
        cp.wait()
    for cp in write_copies(NCHUNK - 1, 1):
        cp.wait()


@functools.partial(jax.jit, static_argnames=())
def kernel(TE, day_table, week_table):
    te_flat = TE.reshape(N * 5 // 128, 128).astype(jnp.int32)
    run = pl.kernel(
        _body,
        out_type=jax.ShapeDtypeStruct((N, 2, TDIM), jnp.float32),
        mesh=plsc.VectorSubcoreMesh(core_axis_name="c", subcore_axis_name="s"),
        scratch_types=[
            pltpu.VMEM((2, TEROWS, 128), jnp.int32),
            pltpu.VMEM((2, SUB, 128), jnp.int32),
            pltpu.VMEM((2, SUB, 128), jnp.int32),
            pltpu.VMEM((2, SUB, 128, TDIM), jnp.float32),
            pltpu.VMEM((2, SUB, 128, TDIM), jnp.float32),
            pltpu.SemaphoreType.DMA((2,)),
            pltpu.SemaphoreType.DMA((2,)),
            pltpu.SemaphoreType.DMA((2,)),
        ],
        compiler_params=pltpu.CompilerParams(
            use_tc_tiling_on_sc=False, needs_layout_passes=False),
    )
    out = run(te_flat, day_table, week_table)
    return out.reshape(B, T, 2 * TDIM)


# speedup vs baseline: 3.1293x; 2.6563x over previous
import functools

import jax
import jax.numpy as jnp
from jax import lax
from jax.experimental import pallas as pl
from jax.experimental.pallas import tpu as pltpu
from jax.experimental.pallas import tpu_sc as plsc

B, T, TDIM = 4096, 200, 64
N = B * T
NC, NS, L = 2, 16, 16
NW = NC * NS
NPW = N // NW
CHUNK = 256
NCHUNK = NPW // CHUNK
GROUPS = CHUNK // L
TEROWS = CHUNK * 5 // 128
DPAD = TDIM + 1
SPAD = 2 * TDIM + 5


def _body(te_hbm, dayp_hbm, weekp_hbm, out_hbm,
          day_t, week_t, te_v, stage_v, tsem, osem, lsem):
    wid = lax.axis_index("s") * NC + lax.axis_index("c")
    lane = jax.lax.iota(jnp.int32, L)
    lane5 = lane * 5
    chunk0 = wid * NCHUNK

    pltpu.async_copy(dayp_hbm, day_t, lsem).wait()
    pltpu.async_copy(weekp_hbm, week_t, lsem).wait()

    def te_copy(k, slot):
        return pltpu.make_async_copy(
            te_hbm.at[pl.ds((chunk0 + k) * TEROWS, TEROWS)],
            te_v.at[slot], tsem.at[slot])

    def out_copy(k, slot):
        return pltpu.make_async_copy(
            stage_v.at[slot, :, pl.ds(0, 2 * TDIM)],
            out_hbm.at[pl.ds((chunk0 + k) * CHUNK, CHUNK)],
            osem.at[slot])

    te_copy(0, 0).start()

    def chunk_step(k, slot):
        te_copy(k, slot).wait()

        @pl.when(k + 1 < NCHUNK)
        def _prefetch():
            te_copy(k + 1, slot ^ 1).start()

        @pl.when(k >= 2)
        def _drain_out():
            out_copy(k - 2, slot).wait()

        def group_step(g, _):
            i5 = lane5 + g * (L * 5)

            def col(off):
                idx = i5 + off
                return plsc.load_gather(te_v.at[slot], [idx >> 7, idx & 127])

            w = col(2)
            h = col(3)
            m = col(4)
            week = w % 7
            day = ((h % 24) * 60 + (m % 60)) // 5
            rows = lane + g * L
            for c in range(TDIM):
                cc = jnp.full((L,), c, jnp.int32)
                wv = plsc.load_gather(week_t, [week, cc])
                plsc.store_scatter(stage_v.at[slot], [rows, cc], wv)
                dv = plsc.load_gather(day_t, [day, cc])
                plsc.store_scatter(
                    stage_v.at[slot], [rows, cc + TDIM], dv)
            return ()

        lax.fori_loop(0, GROUPS, group_step, (), unroll=False)
        out_copy(k, slot).start()

    def pair_step(k2, _):
        chunk_step(2 * k2, 0)
        chunk_step(2 * k2 + 1, 1)
        return ()

    lax.fori_loop(0, NCHUNK // 2, pair_step, (), unroll=False)

    out_copy(NCHUNK - 2, 0).wait()
    out_copy(NCHUNK - 1, 1).wait()


@functools.partial(jax.jit, static_argnames=())
def kernel(TE, day_table, week_table):
    te_flat = TE.reshape(N * 5 // 128, 128).astype(jnp.int32)
    dayp = jnp.pad(day_table, ((0, 0), (0, DPAD - TDIM)))
    weekp = jnp.pad(week_table, ((0, 0), (0, DPAD - TDIM)))
    run = pl.kernel(
        _body,
        out_type=jax.ShapeDtypeStruct((N, 2 * TDIM), jnp.float32),
        mesh=plsc.VectorSubcoreMesh(core_axis_name="c", subcore_axis_name="s"),
        scratch_types=[
            pltpu.VMEM((288, DPAD), jnp.float32),
            pltpu.VMEM((7, DPAD), jnp.float32),
            pltpu.VMEM((2, TEROWS, 128), jnp.int32),
            pltpu.VMEM((2, CHUNK, SPAD), jnp.float32),
            pltpu.SemaphoreType.DMA((2,)),
            pltpu.SemaphoreType.DMA((2,)),
            pltpu.SemaphoreType.DMA,
        ],
        compiler_params=pltpu.CompilerParams(
            use_tc_tiling_on_sc=False, needs_layout_passes=False),
    )
    out = run(te_flat, dayp, weekp)
    return out.reshape(B, T, 2 * TDIM)


# speedup vs baseline: 3.3076x; 1.0570x over previous
import functools

import jax
import jax.numpy as jnp
from jax import lax
from jax.experimental import pallas as pl
from jax.experimental.pallas import tpu as pltpu
from jax.experimental.pallas import tpu_sc as plsc

B, T, TDIM = 4096, 200, 64
N = B * T
NC, NS, L = 2, 16, 16
NW = NC * NS
NPW = N // NW
CHUNK = 256
NCHUNK = NPW // CHUNK
GROUPS = CHUNK // L
TEROWS = CHUNK * 5 // 128
DPAD = TDIM + 1
SPAD = 2 * TDIM + 5


def _body(te_hbm, dayp_hbm, weekp_hbm, out_hbm,
          day_t, week_t, te_v, stage_v, tsem, osem, lsem):
    wid = lax.axis_index("s") * NC + lax.axis_index("c")
    lane = jax.lax.iota(jnp.int32, L)
    lane5 = lane * 5
    chunk0 = wid * NCHUNK

    pltpu.async_copy(dayp_hbm, day_t, lsem).wait()
    pltpu.async_copy(weekp_hbm, week_t, lsem).wait()

    def te_copy(k, slot):
        return pltpu.make_async_copy(
            te_hbm.at[pl.ds((chunk0 + k) * TEROWS, TEROWS)],
            te_v.at[slot], tsem.at[slot])

    def out_copy(k, slot):
        return pltpu.make_async_copy(
            stage_v.at[slot, :, pl.ds(0, 2 * TDIM)],
            out_hbm.at[pl.ds((chunk0 + k) * CHUNK, CHUNK)],
            osem.at[slot])

    te_copy(0, 0).start()

    def chunk_step(k, slot):
        te_copy(k, slot).wait()

        @pl.when(k + 1 < NCHUNK)
        def _prefetch():
            te_copy(k + 1, slot ^ 1).start()

        @pl.when(k >= 2)
        def _drain_out():
            out_copy(k - 2, slot).wait()

        @plsc.parallel_loop(0, GROUPS, unroll=2)
        def group_step(g):
            i5 = lane5 + g * (L * 5)

            def col(off):
                idx = i5 + off
                return plsc.load_gather(te_v.at[slot], [idx >> 7, idx & 127])

            w = col(2)
            h = col(3)
            m = col(4)
            week = w % 7
            day = ((h % 24) * 60 + (m % 60)) // 5
            rows = lane + g * L
            stage = stage_v.at[slot]
            for c in range(TDIM):
                cc = jnp.full((L,), c, jnp.int32)
                wv = plsc.load_gather(week_t, [week, cc])
                plsc.store_scatter(stage, [rows, cc], wv)
                dv = plsc.load_gather(day_t, [day, cc])
                plsc.store_scatter(stage, [rows, cc + TDIM], dv)
        out_copy(k, slot).start()

    def pair_step(k2, _):
        chunk_step(2 * k2, 0)
        chunk_step(2 * k2 + 1, 1)
        return ()

    lax.fori_loop(0, NCHUNK // 2, pair_step, (), unroll=False)

    out_copy(NCHUNK - 2, 0).wait()
    out_copy(NCHUNK - 1, 1).wait()


@functools.partial(jax.jit, static_argnames=())
def kernel(TE, day_table, week_table):
    te_flat = TE.reshape(N * 5 // 128, 128).astype(jnp.int32)
    dayp = jnp.pad(day_table, ((0, 0), (0, DPAD - TDIM)))
    weekp = jnp.pad(week_table, ((0, 0), (0, DPAD - TDIM)))
    run = pl.kernel(
        _body,
        out_type=jax.ShapeDtypeStruct((N, 2 * TDIM), jnp.float32),
        mesh=plsc.VectorSubcoreMesh(core_axis_name="c", subcore_axis_name="s"),
        scratch_types=[
            pltpu.VMEM((288, DPAD), jnp.float32),
            pltpu.VMEM((7, DPAD), jnp.float32),
            pltpu.VMEM((2, TEROWS, 128), jnp.int32),
            pltpu.VMEM((2, CHUNK, SPAD), jnp.float32),
            pltpu.SemaphoreType.DMA((2,)),
            pltpu.SemaphoreType.DMA((2,)),
            pltpu.SemaphoreType.DMA,
        ],
        compiler_params=pltpu.CompilerParams(
            use_tc_tiling_on_sc=False, needs_layout_passes=False),
    )
    out = run(te_flat, dayp, weekp)
    return out.reshape(B, T, 2 * TDIM)


# speedup vs baseline: 4.9673x; 1.5018x over previous
import functools

import jax
import jax.numpy as jnp
from jax import lax
from jax.experimental import pallas as pl
from jax.experimental.pallas import tpu as pltpu
from jax.experimental.pallas import tpu_sc as plsc

B, T, TDIM = 4096, 200, 64
N = B * T
NC, NS, L = 2, 16, 16
NW = NC * NS
NPW = N // NW
CHUNK = 256
NCHUNK = NPW // CHUNK
GROUPS = CHUNK // L
TEROWS = CHUNK * 5 // 128
SEG = TDIM // L


def _body(te_hbm, day_hbm, week_hbm, out_hbm,
          day_t, week_t, te_v, stage_v, tsem, osem, lsem):
    wid = lax.axis_index("s") * NC + lax.axis_index("c")
    lane5 = jax.lax.iota(jnp.int32, L) * 5
    chunk0 = wid * NCHUNK

    pltpu.async_copy(day_hbm, day_t, lsem).wait()
    pltpu.async_copy(week_hbm, week_t, lsem).wait()

    def te_copy(k, slot):
        return pltpu.make_async_copy(
            te_hbm.at[pl.ds((chunk0 + k) * TEROWS, TEROWS)],
            te_v.at[slot], tsem.at[slot])

    def out_copy(k, slot):
        return pltpu.make_async_copy(
            stage_v.at[slot],
            out_hbm.at[pl.ds((chunk0 + k) * CHUNK, CHUNK)],
            osem.at[slot])

    te_copy(0, 0).start()

    def chunk_step(k, slot):
        te_copy(k, slot).wait()

        @pl.when(k + 1 < NCHUNK)
        def _prefetch():
            te_copy(k + 1, slot ^ 1).start()

        @pl.when(k >= 2)
        def _drain_out():
            out_copy(k - 2, slot).wait()

        @plsc.parallel_loop(0, GROUPS, unroll=2)
        def group_step(g):
            i5 = lane5 + g * (L * 5)

            def col(off):
                idx = i5 + off
                return plsc.load_gather(te_v.at[slot], [idx >> 7, idx & 127])

            w = col(2)
            h = col(3)
            m = col(4)
            week = w % 7
            day = ((h % 24) * 60 + (m % 60)) // 5
            for e in range(L):
                we = week[e]
                de = day[e]
                row = g * L + e
                for s in range(SEG):
                    stage_v[slot, row, pl.ds(s * L, L)] = \
                        week_t[we, pl.ds(s * L, L)]
                    stage_v[slot, row, pl.ds(TDIM + s * L, L)] = \
                        day_t[de, pl.ds(s * L, L)]

        out_copy(k, slot).start()

    def pair_step(k2, _):
        chunk_step(2 * k2, 0)
        chunk_step(2 * k2 + 1, 1)
        return ()

    lax.fori_loop(0, NCHUNK // 2, pair_step, (), unroll=False)

    out_copy(NCHUNK - 2, 0).wait()
    out_copy(NCHUNK - 1, 1).wait()


@functools.partial(jax.jit, static_argnames=())
def kernel(TE, day_table, week_table):
    te_flat = TE.reshape(N * 5 // 128, 128).astype(jnp.int32)
    run = pl.kernel(
        _body,
        out_type=jax.ShapeDtypeStruct((N, 2 * TDIM), jnp.float32),
        mesh=plsc.VectorSubcoreMesh(core_axis_name="c", subcore_axis_name="s"),
        scratch_types=[
            pltpu.VMEM((288, TDIM), jnp.float32),
            pltpu.VMEM((7, TDIM), jnp.float32),
            pltpu.VMEM((2, TEROWS, 128), jnp.int32),
            pltpu.VMEM((2, CHUNK, 2 * TDIM), jnp.float32),
            pltpu.SemaphoreType.DMA((2,)),
            pltpu.SemaphoreType.DMA((2,)),
            pltpu.SemaphoreType.DMA,
        ],
        compiler_params=pltpu.CompilerParams(
            use_tc_tiling_on_sc=False, needs_layout_passes=False),
    )
    out = run(te_flat, day_table, week_table)
    return out.reshape(B, T, 2 * TDIM)


# speedup vs baseline: 8.7744x; 1.7664x over previous
import functools

import jax
import jax.numpy as jnp
from jax import lax
from jax.experimental import pallas as pl
from jax.experimental.pallas import tpu as pltpu
from jax.experimental.pallas import tpu_sc as plsc

B, T, TDIM = 4096, 200, 64
N = B * T
NC, NS, L = 2, 16, 16
NW = NC * NS
NPW = N // NW
CHUNK = 128
NCHUNK = NPW // CHUNK
GROUPS = CHUNK // L
TEROWS = CHUNK * 5 // 128


def _body(te_hbm, day_hbm, week_hbm, out_hbm,
          day_t, week_t, te_v, widx_v, didx_v, wrows_v, drows_v,
          tsem, gsem, osem, lsem):
    sid = lax.axis_index("s")
    wid = sid * NC + lax.axis_index("c")
    lane5 = jax.lax.iota(jnp.int32, L) * 5
    chunk0 = wid * NCHUNK

    @pl.when(sid == 0)
    def _load_tables():
        pltpu.async_copy(day_hbm, day_t, lsem).wait()
        pltpu.async_copy(week_hbm, week_t, lsem).wait()

    plsc.subcore_barrier()

    def te_copy(k, slot):
        return pltpu.make_async_copy(
            te_hbm.at[pl.ds((chunk0 + k) * TEROWS, TEROWS)],
            te_v.at[slot], tsem.at[slot])

    def gathers(slot):
        return (pltpu.make_async_copy(
                    week_t.at[widx_v.at[slot]], wrows_v.at[slot],
                    gsem.at[slot]),
                pltpu.make_async_copy(
                    day_t.at[didx_v.at[slot]], drows_v.at[slot],
                    gsem.at[slot]))

    def out_copies(k, slot):
        rows = pl.ds((chunk0 + k) * CHUNK, CHUNK)
        return (pltpu.make_async_copy(
                    wrows_v.at[slot], out_hbm.at[rows, 0], osem.at[slot]),
                pltpu.make_async_copy(
                    drows_v.at[slot], out_hbm.at[rows, 1], osem.at[slot]))

    te_copy(0, 0).start()

    def chunk_step(k, slot):
        te_copy(k, slot).wait()

        @pl.when(k + 1 < NCHUNK)
        def _prefetch():
            te_copy(k + 1, slot ^ 1).start()

        @pl.when(k >= 2)
        def _drain_out():
            ow, od = out_copies(k - 2, slot)
            ow.wait()
            od.wait()

        @plsc.parallel_loop(0, GROUPS, unroll=2)
        def group_step(g):
            i5 = lane5 + g * (L * 5)

            def col(off):
                idx = i5 + off
                return plsc.load_gather(te_v.at[slot], [idx >> 7, idx & 127])

            w = col(2)
            h = col(3)
            m = col(4)
            widx_v[slot, pl.ds(g * L, L)] = w % 7
            didx_v[slot, pl.ds(g * L, L)] = ((h % 24) * 60 + (m % 60)) // 5

        gw, gd = gathers(slot)
        gw.start()
        gd.start()

        @pl.when(k >= 1)
        def _flush_prev():
            pw, pd = gathers(slot ^ 1)
            pw.wait()
            pd.wait()
            ow, od = out_copies(k - 1, slot ^ 1)
            ow.start()
            od.start()

    def pair_step(k2, _):
        chunk_step(2 * k2, 0)
        chunk_step(2 * k2 + 1, 1)
        return ()

    lax.fori_loop(0, NCHUNK // 2, pair_step, (), unroll=False)

    last = NCHUNK - 1
    gw, gd = gathers(last & 1)
    gw.wait()
    gd.wait()
    ow, od = out_copies(last, last & 1)
    ow.start()
    od.start()
    for k in (NCHUNK - 2, NCHUNK - 1):
        ow, od = out_copies(k, k & 1)
        ow.wait()
        od.wait()


@functools.partial(jax.jit, static_argnames=())
def kernel(TE, day_table, week_table):
    te_flat = TE.reshape(N * 5 // 128, 128).astype(jnp.int32)
    run = pl.kernel(
        _body,
        out_type=jax.ShapeDtypeStruct((N, 2, TDIM), jnp.float32),
        mesh=plsc.VectorSubcoreMesh(core_axis_name="c", subcore_axis_name="s"),
        scratch_types=[
            pltpu.VMEM_SHARED((288, TDIM), jnp.float32),
            pltpu.VMEM_SHARED((7, TDIM), jnp.float32),
            pltpu.VMEM((2, TEROWS, 128), jnp.int32),
            pltpu.VMEM((2, CHUNK), jnp.int32),
            pltpu.VMEM((2, CHUNK), jnp.int32),
            pltpu.VMEM((2, CHUNK, TDIM), jnp.float32),
            pltpu.VMEM((2, CHUNK, TDIM), jnp.float32),
            pltpu.SemaphoreType.DMA((2,)),
            pltpu.SemaphoreType.DMA((2,)),
            pltpu.SemaphoreType.DMA((2,)),
            pltpu.SemaphoreType.DMA,
        ],
        compiler_params=pltpu.CompilerParams(
            use_tc_tiling_on_sc=False, needs_layout_passes=False),
    )
    out = run(te_flat, day_table, week_table)
    return out.reshape(B, T, 2 * TDIM)
